# baseline (speedup 1.0000x reference)
import functools

import jax
import jax.numpy as jnp
from jax import lax
from jax.experimental import pallas as pl
from jax.experimental.pallas import tpu as pltpu
from jax.experimental.pallas import tpu_sc as plsc

NUM_CLASSES = 1000000
D_MODEL = 64
BATCH = 4096
SEQ = 200

_NC = 2
_NS = 16
_NW = _NC * _NS

_CHUNK = 128
_DPAD = 128

_TBLK = 4096
_TGRID = (NUM_CLASSES + _TBLK - 1) // _TBLK


def _prep_body(tt_ref, out_ref):
  x = tt_ref[...].T
  out_ref[...] = jnp.concatenate(
      [x, jnp.zeros((_TBLK, _DPAD - D_MODEL), jnp.float32)], axis=1)


@jax.jit
def _prep(table_t):
  return pl.pallas_call(
      _prep_body,
      grid=(_TGRID,),
      in_specs=[pl.BlockSpec((D_MODEL, _TBLK), lambda j: (0, j))],
      out_specs=pl.BlockSpec((_TBLK, _DPAD), lambda j: (j, 0)),
      out_shape=jax.ShapeDtypeStruct((NUM_CLASSES, _DPAD), jnp.float32),
  )(table_t)


def _gather_body(idx_hbm, table_hbm, out_hbm, idx_v, gbufs, tbufs, gsems,
                 osems):
  wid = lax.axis_index("s") * _NC + lax.axis_index("c")
  col0 = wid * _CHUNK

  pltpu.sync_copy(idx_hbm.at[:, pl.ds(col0, _CHUNK)], idx_v)

  def start_gather(s, b):
    pltpu.async_copy(table_hbm.at[idx_v.at[s]], gbufs.at[b], gsems.at[b])

  def wait_gather(s, b):
    pltpu.make_async_copy(table_hbm.at[idx_v.at[s]], gbufs.at[b],
                          gsems.at[b]).wait()

  def start_write(s, b):
    pltpu.async_copy(tbufs.at[b], out_hbm.at[s, :, pl.ds(col0, _CHUNK)],
                     osems.at[b])

  def wait_write(s, b):
    pltpu.make_async_copy(tbufs.at[b], out_hbm.at[s, :, pl.ds(col0, _CHUNK)],
                          osems.at[b]).wait()

  iota = lax.iota(jnp.int32, 16)

  def transpose(b):
    gb = gbufs.at[b]
    tb = tbufs.at[b]

    @pl.loop(0, D_MODEL)
    def _(d):
      dvec = jnp.zeros((16,), jnp.int32) + d
      for j0 in range(0, _CHUNK, 16):
        v = plsc.load_gather(gb, [iota + j0, dvec])
        tb[d, pl.ds(j0, 16)] = v

  start_gather(0, 0)
  start_gather(1, 1)

  @pl.loop(0, SEQ, step=2)
  def _(s0):
    for b in range(2):
      s = s0 + b
      wait_gather(s, b)

      @pl.when(s >= 2)
      def _():
        wait_write(s - 2, b)

      transpose(b)
      start_write(s, b)

      @pl.when(s + 2 < SEQ)
      def _():
        start_gather(s + 2, b)

  wait_write(SEQ - 2, 0)
  wait_write(SEQ - 1, 1)


@jax.jit
def _run(idx_t, table_padded):
  f = pl.kernel(
      _gather_body,
      out_type=jax.ShapeDtypeStruct((SEQ, D_MODEL, BATCH), jnp.float32),
      mesh=plsc.VectorSubcoreMesh(core_axis_name="c", subcore_axis_name="s"),
      compiler_params=pltpu.CompilerParams(needs_layout_passes=False),
      scratch_types=[
          pltpu.VMEM((SEQ, _CHUNK), jnp.int32),
          pltpu.VMEM((2, _CHUNK, _DPAD), jnp.float32),
          pltpu.VMEM((2, D_MODEL, _CHUNK), jnp.float32),
          pltpu.SemaphoreType.DMA((2,)),
          pltpu.SemaphoreType.DMA((2,)),
      ],
  )
  return f(idx_t, table_padded)


def kernel(classes, bbs, class_embedding):
  del bbs
  table_padded = _prep(class_embedding.T)
  out_t = _run(classes.T.astype(jnp.int32), table_padded)
  return out_t.transpose(2, 0, 1)


# speedup vs baseline: 2.0810x; 2.0810x over previous
import functools

import jax
import jax.numpy as jnp
from jax import lax
from jax.experimental import pallas as pl
from jax.experimental.pallas import tpu as pltpu
from jax.experimental.pallas import tpu_sc as plsc

NUM_CLASSES = 1000000
D_MODEL = 64
BATCH = 4096
SEQ = 200

_NC = 2
_NS = 16
_NW = _NC * _NS

_B = BATCH * SEQ
_CHUNK = 128
_PER_W = _B // _NW
_NCHUNK = _PER_W // _CHUNK
_NBUF = 5
_DPAD = 128

_TBLK = 8192
_TGRID = (NUM_CLASSES + _TBLK - 1) // _TBLK


def _prep_body(tt_ref, out_ref):
  x = tt_ref[...].T
  out_ref[...] = jnp.concatenate(
      [x, jnp.zeros((_TBLK, _DPAD - D_MODEL), jnp.float32)], axis=1)


@jax.jit
def _prep(table_t):
  return pl.pallas_call(
      _prep_body,
      grid=(_TGRID,),
      in_specs=[pl.BlockSpec((D_MODEL, _TBLK), lambda j: (0, j))],
      out_specs=pl.BlockSpec((_TBLK, _DPAD), lambda j: (j, 0)),
      out_shape=jax.ShapeDtypeStruct((NUM_CLASSES, _DPAD), jnp.float32),
  )(table_t)


def _gather_body(idx_hbm, table_hbm, out_hbm, idx_v, bufs, gsems, osems):
  wid = lax.axis_index("s") * _NC + lax.axis_index("c")
  base = wid * _PER_W

  pltpu.sync_copy(idx_hbm.at[wid], idx_v)

  def start_gather(c, b):
    pltpu.async_copy(table_hbm.at[idx_v.at[c]], bufs.at[b], gsems.at[b])

  def wait_gather(c, b):
    pltpu.make_async_copy(table_hbm.at[idx_v.at[c]], bufs.at[b],
                          gsems.at[b]).wait()

  def start_write(c, b):
    pltpu.async_copy(bufs.at[b],
                     out_hbm.at[pl.ds(base + c * _CHUNK, _CHUNK)],
                     osems.at[b])

  def wait_write(c, b):
    pltpu.make_async_copy(bufs.at[b],
                          out_hbm.at[pl.ds(base + c * _CHUNK, _CHUNK)],
                          osems.at[b]).wait()

  for b in range(_NBUF):
    start_gather(b, b)

  @pl.loop(0, _NCHUNK, step=_NBUF)
  def _(c0):
    for b in range(_NBUF):
      c = c0 + b
      wait_gather(c, b)
      start_write(c, b)
      pb = (b - 1) % _NBUF
      pc = c - 1
      nxt = pc + _NBUF

      @pl.when(jnp.logical_and(pc >= 0, nxt < _NCHUNK))
      def _():
        wait_write(pc, pb)
        start_gather(nxt, pb)

  for b in range(_NBUF):
    c = _NCHUNK - _NBUF + b
    wait_write(c, c % _NBUF)


@jax.jit
def _run(classes_flat, table_padded):
  idx3 = classes_flat.reshape(_NW, _NCHUNK, _CHUNK)
  f = pl.kernel(
      _gather_body,
      out_type=jax.ShapeDtypeStruct((_B, _DPAD), jnp.float32),
      mesh=plsc.VectorSubcoreMesh(core_axis_name="c", subcore_axis_name="s"),
      scratch_types=[
          pltpu.VMEM((_NCHUNK, _CHUNK), jnp.int32),
          pltpu.VMEM((_NBUF, _CHUNK, _DPAD), jnp.float32),
          pltpu.SemaphoreType.DMA((_NBUF,)),
          pltpu.SemaphoreType.DMA((_NBUF,)),
      ],
  )
  return f(idx3, table_padded)


def kernel(classes, bbs, class_embedding):
  del bbs
  table_padded = _prep(class_embedding.T)
  out = _run(classes.reshape(-1).astype(jnp.int32), table_padded)
  return out[:, :D_MODEL].reshape(BATCH, SEQ, D_MODEL)


# speedup vs baseline: 2.1364x; 1.0266x over previous
import functools

import jax
import jax.numpy as jnp
from jax import lax
from jax.experimental import pallas as pl
from jax.experimental.pallas import tpu as pltpu
from jax.experimental.pallas import tpu_sc as plsc

NUM_CLASSES = 1000000
D_MODEL = 64
BATCH = 4096
SEQ = 200

_NC = 2
_NS = 16
_NW = _NC * _NS

_B = BATCH * SEQ
_CHUNK = 128
_PER_W = _B // _NW
_NCHUNK = _PER_W // _CHUNK
_NBUF = 5
_DPAD = 128

_TBLK = 16384
_TGRID = (NUM_CLASSES + _TBLK - 1) // _TBLK


def _prep_body(tt_ref, out_ref):
  x = tt_ref[...].T
  out_ref[...] = jnp.concatenate(
      [x, jnp.zeros((_TBLK, _DPAD - D_MODEL), jnp.float32)], axis=1)


@jax.jit
def _prep(table_t):
  return pl.pallas_call(
      _prep_body,
      grid=(_TGRID,),
      in_specs=[pl.BlockSpec((D_MODEL, _TBLK), lambda j: (0, j))],
      out_specs=pl.BlockSpec((_TBLK, _DPAD), lambda j: (j, 0)),
      out_shape=jax.ShapeDtypeStruct((NUM_CLASSES, _DPAD), jnp.float32),
  )(table_t)


def _gather_body(idx_hbm, table_hbm, out_hbm, idx_v, bufs, gsems, osems):
  wid = lax.axis_index("s") * _NC + lax.axis_index("c")
  base = wid * _PER_W

  pltpu.sync_copy(idx_hbm.at[wid], idx_v)

  def start_gather(c, b):
    pltpu.async_copy(table_hbm.at[idx_v.at[c]], bufs.at[b], gsems.at[b])

  def wait_gather(c, b):
    pltpu.make_async_copy(table_hbm.at[idx_v.at[c]], bufs.at[b],
                          gsems.at[b]).wait()

  def start_write(c, b):
    pltpu.async_copy(bufs.at[b],
                     out_hbm.at[pl.ds(base + c * _CHUNK, _CHUNK)],
                     osems.at[b])

  def wait_write(c, b):
    pltpu.make_async_copy(bufs.at[b],
                          out_hbm.at[pl.ds(base + c * _CHUNK, _CHUNK)],
                          osems.at[b]).wait()

  for b in range(_NBUF):
    start_gather(b, b)

  @pl.loop(0, _NCHUNK, step=_NBUF)
  def _(c0):
    for b in range(_NBUF):
      c = c0 + b
      wait_gather(c, b)
      start_write(c, b)
      pb = (b - 1) % _NBUF
      pc = c - 1
      nxt = pc + _NBUF

      @pl.when(jnp.logical_and(pc >= 0, nxt < _NCHUNK))
      def _():
        wait_write(pc, pb)
        start_gather(nxt, pb)

  for b in range(_NBUF):
    c = _NCHUNK - _NBUF + b
    wait_write(c, c % _NBUF)


@jax.jit
def _run(classes_flat, table_padded):
  idx3 = classes_flat.reshape(_NW, _NCHUNK, _CHUNK)
  f = pl.kernel(
      _gather_body,
      out_type=jax.ShapeDtypeStruct((_B, _DPAD), jnp.float32),
      mesh=plsc.VectorSubcoreMesh(core_axis_name="c", subcore_axis_name="s"),
      scratch_types=[
          pltpu.VMEM((_NCHUNK, _CHUNK), jnp.int32),
          pltpu.VMEM((_NBUF, _CHUNK, _DPAD), jnp.float32),
          pltpu.SemaphoreType.DMA((_NBUF,)),
          pltpu.SemaphoreType.DMA((_NBUF,)),
      ],
  )
  return f(idx3, table_padded)


def kernel(classes, bbs, class_embedding):
  del bbs
  table_padded = _prep(class_embedding.T)
  out = _run(classes.reshape(-1).astype(jnp.int32), table_padded)
  return out[:, :D_MODEL].reshape(BATCH, SEQ, D_MODEL)
